# baseline (speedup 1.0000x reference)
import math

import jax
import jax.numpy as jnp
import numpy as np
from jax.experimental import pallas as pl
from jax.experimental.pallas import tpu as pltpu
from jax.experimental.pallas import tpu_sc as plsc
import functools

D_MODEL = 768
N_HEADS = 12
DH = D_MODEL // N_HEADS
D_FF = 3072
HORIZON = 22
HQ = 24
B_SZ = 4
S_LEN = 8192
NQ = N_HEADS * HQ
U_TOP = min(HORIZON, 5 * int(math.ceil(math.log(HORIZON))))
S_BLK = 2048
NS = S_LEN // S_BLK


def _make_pe_pad():
    pos = np.arange(HORIZON, dtype=np.float32)[:, None]
    div = np.exp(np.arange(0, D_MODEL, 2, dtype=np.float32)
                 * (-math.log(10000.0) / D_MODEL))
    pe = np.zeros((HQ, D_MODEL), dtype=np.float32)
    pe[:HORIZON, 0::2] = np.sin(pos * div)
    pe[:HORIZON, 1::2] = np.cos(pos * div)
    return pe


def _eye(n):
    ii = jax.lax.broadcasted_iota(jnp.int32, (n, n), 0)
    jj = jax.lax.broadcasted_iota(jnp.int32, (n, n), 1)
    return (ii == jj).astype(jnp.float32)


def _to_col(row):
    n = row.shape[1]
    return jax.lax.dot_general(_eye(n), row, (((1,), (1,)), ((), ())),
                               precision=jax.lax.Precision.HIGHEST,
                               preferred_element_type=jnp.float32)


def _score_body(pe_ref, wq_ref, bq_ref, enc_ref, wk_ref,
                sc_ref, m_ref, l_ref, ssum_ref, qp_scr):
    b = pl.program_id(0)
    s = pl.program_id(1)
    scale = 1.0 / math.sqrt(DH)

    @pl.when((b == 0) & (s == 0))
    def _build_qp():
        qp = jax.lax.dot_general(pe_ref[...], wq_ref[...],
                                 (((1,), (1,)), ((), ())),
                                 preferred_element_type=jnp.float32
                                 ) + bq_ref[...]
        lanes = jax.lax.broadcasted_iota(jnp.int32, (1, D_MODEL), 1)
        blocks = []
        for h in range(N_HEADS):
            msk = ((lanes >= h * DH)
                   & (lanes < (h + 1) * DH)).astype(jnp.float32)
            blocks.append(qp * msk)
        qp_scr[...] = jnp.concatenate(blocks, axis=0)

    enc = enc_ref[0]
    k_blk = jax.lax.dot_general(enc, wk_ref[...], (((1,), (1,)), ((), ())),
                                preferred_element_type=jnp.float32)
    scores = jax.lax.dot_general(k_blk, qp_scr[...], (((1,), (1,)), ((), ())),
                                 preferred_element_type=jnp.float32)
    sc_ref[0] = scores
    blk_max = jnp.max(scores, axis=0, keepdims=True)
    blk_sum = jnp.sum(scores, axis=0, keepdims=True)

    @pl.when(s == 0)
    def _init():
        m_ref[0] = blk_max
        l_ref[0] = jnp.sum(jnp.exp((scores - blk_max) * scale),
                           axis=0, keepdims=True)
        ssum_ref[0] = blk_sum

    @pl.when(s != 0)
    def _update():
        m_old = m_ref[0]
        m_new = jnp.maximum(m_old, blk_max)
        alpha = jnp.exp((m_old - m_new) * scale)
        p = jnp.exp((scores - m_new) * scale)
        m_ref[0] = m_new
        l_ref[0] = l_ref[0] * alpha + jnp.sum(p, axis=0, keepdims=True)
        ssum_ref[0] = ssum_ref[0] + blk_sum


def _attn_body(enc_ref, sc_ref, m_ref, l_ref, wv_ref, bv_ref,
               vsum_ref, pacc_ref):
    s = pl.program_id(1)
    enc = enc_ref[0]
    v_blk = jax.lax.dot_general(enc, wv_ref[...], (((1,), (1,)), ((), ())),
                                preferred_element_type=jnp.float32) + bv_ref[...]
    scale = 1.0 / math.sqrt(DH)
    w = jnp.exp((sc_ref[0] - m_ref[0]) * scale) / l_ref[0]
    vsum = jnp.sum(v_blk, axis=0, keepdims=True)
    pacc = jax.lax.dot_general(w, v_blk, (((0,), (0,)), ((), ())),
                               preferred_element_type=jnp.float32)

    @pl.when(s == 0)
    def _init():
        vsum_ref[0] = vsum
        pacc_ref[0] = pacc

    @pl.when(s != 0)
    def _update():
        vsum_ref[0] = vsum_ref[0] + vsum
        pacc_ref[0] = pacc_ref[0] + pacc


NG = B_SZ * N_HEADS
GQ = 32
NCHUNK = NG // 16


def _sc_lazy_select(m_flat, ssum_flat):
    mesh = plsc.VectorSubcoreMesh(core_axis_name="c", subcore_axis_name="s")
    n_in = NCHUNK * GQ * 16
    n_slot = GQ * 16

    @functools.partial(
        pl.kernel, mesh=mesh,
        out_type=jax.ShapeDtypeStruct((32 * n_slot,), jnp.float32),
        scratch_types=[
            pltpu.VMEM((n_in,), jnp.float32),
            pltpu.VMEM((n_in,), jnp.float32),
            pltpu.VMEM((n_slot,), jnp.float32),
            pltpu.VMEM((n_slot,), jnp.float32),
        ],
    )
    def lazy_kernel(m_hbm, s_hbm, out_hbm, m_v, s_v, mst_v, out_v):
        wid = jax.lax.axis_index("s") * 2 + jax.lax.axis_index("c")
        chunk = jax.lax.rem(wid, NCHUNK)
        base = chunk * n_slot
        pltpu.sync_copy(m_hbm, m_v)
        pltpu.sync_copy(s_hbm, s_v)
        inv_s = 1.0 / float(S_LEN)
        for i in range(HORIZON):
            mst_v[pl.ds(i * 16, 16)] = (m_v[pl.ds(base + i * 16, 16)]
                                        - s_v[pl.ds(base + i * 16, 16)] * inv_s)
        zero16 = jnp.zeros((16,), jnp.float32)
        one16 = jnp.ones((16,), jnp.float32)
        for i in range(HORIZON):
            mi = mst_v[pl.ds(i * 16, 16)]
            rank = zero16
            for j in range(HORIZON):
                if j == i:
                    continue
                mj = mst_v[pl.ds(j * 16, 16)]
                beats = (mj >= mi) if j < i else (mj > mi)
                rank = rank + jnp.where(beats, one16, zero16)
            out_v[pl.ds(i * 16, 16)] = jnp.where(rank >= float(U_TOP),
                                                 one16, zero16)
        for i in range(HORIZON, GQ):
            out_v[pl.ds(i * 16, 16)] = zero16
        pltpu.sync_copy(out_v, out_hbm.at[pl.ds(wid * n_slot, n_slot)])

    return lazy_kernel(m_flat, ssum_flat)


def _layer_norm(x, g, b, eps=1e-5):
    m = jnp.mean(x, axis=-1, keepdims=True)
    v = jnp.mean((x - m) ** 2, axis=-1, keepdims=True)
    return (x - m) / jnp.sqrt(v + eps) * g + b


def _tail_body(lazy_ref, m_ref, ssum_ref, vsum_ref, pacc_ref,
               wo_ref, bo_ref, w1_ref, b1_ref, w2_ref, b2_ref,
               g1_ref, be1_ref, g2_ref, be2_ref, wout_ref,
               fc_ref, y_ref):
    del m_ref, ssum_ref
    lazy3 = jnp.concatenate([_to_col(lazy_ref[b])[None] for b in range(B_SZ)],
                            axis=0)

    ctx_all = pacc_ref[...]

    vmean3 = jnp.concatenate([vsum_ref[b][None] for b in range(B_SZ)],
                             axis=0) / S_LEN

    lanes = jax.lax.broadcasted_iota(jnp.int32, (1, 1, D_MODEL), 2)
    attn = jnp.zeros((B_SZ, HQ, D_MODEL), jnp.float32)
    for h in range(N_HEADS):
        mask = ((lanes >= h * DH) & (lanes < (h + 1) * DH)).astype(jnp.float32)
        act = ctx_all[:, h * HQ:(h + 1) * HQ, :]
        lz = lazy3[:, h * HQ:(h + 1) * HQ, :]
        sel = act * (1.0 - lz) + vmean3 * lz
        attn = attn + sel * mask
    attn = attn.reshape(B_SZ * HQ, D_MODEL)

    proj = jax.lax.dot_general(attn, wo_ref[...], (((1,), (1,)), ((), ())),
                               preferred_element_type=jnp.float32) + bo_ref[...]
    x = _layer_norm(proj + proj, g1_ref[...], be1_ref[...])
    h1 = jax.lax.dot_general(x, w1_ref[...], (((1,), (1,)), ((), ())),
                             preferred_element_type=jnp.float32) + b1_ref[...]
    h1 = jnp.maximum(h1, 0.0)
    ff = jax.lax.dot_general(h1, w2_ref[...], (((1,), (1,)), ((), ())),
                             preferred_element_type=jnp.float32) + b2_ref[...]
    y = _layer_norm(x + ff, g2_ref[...], be2_ref[...])
    fc = jax.lax.dot_general(y, wout_ref[...], (((1,), (1,)), ((), ())),
                             preferred_element_type=jnp.float32)
    y_ref[...] = y
    fc_ref[...] = fc


def kernel(encoder_output, Wq, bq, Wk, bk, Wv, bv, Wo, bo,
           W1, b1, W2, b2, g1, be1, g2, be2, Wout, bout):
    pe_pad = jnp.asarray(_make_pe_pad())

    def to_chunks(x):
        g = jnp.pad(x.reshape(NG, HQ), ((0, 0), (0, GQ - HQ)))
        return g.T.reshape(GQ, NCHUNK, 16).transpose(1, 0, 2)

    stats_shape = jax.ShapeDtypeStruct((B_SZ, 1, NQ), jnp.float32)
    sc_hbm, m_s, l_s, ssum_s = pl.pallas_call(
        _score_body,
        grid=(B_SZ, NS),
        in_specs=[
            pl.BlockSpec((HQ, D_MODEL), lambda b, s: (0, 0)),
            pl.BlockSpec((D_MODEL, D_MODEL), lambda b, s: (0, 0)),
            pl.BlockSpec((D_MODEL,), lambda b, s: (0,)),
            pl.BlockSpec((1, S_BLK, D_MODEL), lambda b, s: (b, s, 0)),
            pl.BlockSpec((D_MODEL, D_MODEL), lambda b, s: (0, 0)),
        ],
        out_specs=[
            pl.BlockSpec((1, S_BLK, NQ), lambda b, s: (b, s, 0)),
            pl.BlockSpec((1, 1, NQ), lambda b, s: (b, 0, 0)),
            pl.BlockSpec((1, 1, NQ), lambda b, s: (b, 0, 0)),
            pl.BlockSpec((1, 1, NQ), lambda b, s: (b, 0, 0)),
        ],
        out_shape=[
            jax.ShapeDtypeStruct((B_SZ, S_LEN, NQ), jnp.float32),
            stats_shape, stats_shape, stats_shape,
        ],
        scratch_shapes=[pltpu.VMEM((NQ, D_MODEL), jnp.float32)],
    )(pe_pad, Wq, bq, encoder_output, Wk)

    lazy_flat = _sc_lazy_select(to_chunks(m_s).reshape(-1),
                                to_chunks(ssum_s).reshape(-1))
    lazy_chunks = lazy_flat[:NCHUNK * GQ * 16].reshape(NCHUNK, GQ, 16)
    lazy_row = (lazy_chunks.transpose(1, 0, 2).reshape(GQ, NG)
                .T[:, :HQ].reshape(B_SZ, 1, NQ))

    vsum_s, pacc_s = pl.pallas_call(
        _attn_body,
        grid=(B_SZ, NS),
        in_specs=[
            pl.BlockSpec((1, S_BLK, D_MODEL), lambda b, s: (b, s, 0)),
            pl.BlockSpec((1, S_BLK, NQ), lambda b, s: (b, s, 0)),
            pl.BlockSpec((1, 1, NQ), lambda b, s: (b, 0, 0)),
            pl.BlockSpec((1, 1, NQ), lambda b, s: (b, 0, 0)),
            pl.BlockSpec((D_MODEL, D_MODEL), lambda b, s: (0, 0)),
            pl.BlockSpec((D_MODEL,), lambda b, s: (0,)),
        ],
        out_specs=[
            pl.BlockSpec((1, 1, D_MODEL), lambda b, s: (b, 0, 0)),
            pl.BlockSpec((1, NQ, D_MODEL), lambda b, s: (b, 0, 0)),
        ],
        out_shape=[
            jax.ShapeDtypeStruct((B_SZ, 1, D_MODEL), jnp.float32),
            jax.ShapeDtypeStruct((B_SZ, NQ, D_MODEL), jnp.float32),
        ],
    )(encoder_output, sc_hbm, m_s, l_s, Wv, bv)

    fc_pad, y_pad = pl.pallas_call(
        _tail_body,
        out_shape=[
            jax.ShapeDtypeStruct((B_SZ * HQ, 8), jnp.float32),
            jax.ShapeDtypeStruct((B_SZ * HQ, D_MODEL), jnp.float32),
        ],
    )(lazy_row,
      m_s, ssum_s, vsum_s, pacc_s,
      Wo, bo, W1, b1, W2, b2, g1, be1, g2, be2,
      jnp.pad(Wout, ((0, 7), (0, 0))))

    y = y_pad.reshape(B_SZ, HQ, D_MODEL)[:, :HORIZON, :]
    forecasts = fc_pad[:, 0].reshape(B_SZ, HQ)[:, :HORIZON] + bout
    return forecasts, y


# speedup vs baseline: 1.0048x; 1.0048x over previous
import math

import jax
import jax.numpy as jnp
import numpy as np
from jax.experimental import pallas as pl
from jax.experimental.pallas import tpu as pltpu
from jax.experimental.pallas import tpu_sc as plsc
import functools

D_MODEL = 768
N_HEADS = 12
DH = D_MODEL // N_HEADS
D_FF = 3072
HORIZON = 22
HQ = 24
B_SZ = 4
S_LEN = 8192
NQ = N_HEADS * HQ
U_TOP = min(HORIZON, 5 * int(math.ceil(math.log(HORIZON))))
S_BLK = 4096
NS = S_LEN // S_BLK


def _make_pe_pad():
    pos = np.arange(HORIZON, dtype=np.float32)[:, None]
    div = np.exp(np.arange(0, D_MODEL, 2, dtype=np.float32)
                 * (-math.log(10000.0) / D_MODEL))
    pe = np.zeros((HQ, D_MODEL), dtype=np.float32)
    pe[:HORIZON, 0::2] = np.sin(pos * div)
    pe[:HORIZON, 1::2] = np.cos(pos * div)
    return pe


def _eye(n):
    ii = jax.lax.broadcasted_iota(jnp.int32, (n, n), 0)
    jj = jax.lax.broadcasted_iota(jnp.int32, (n, n), 1)
    return (ii == jj).astype(jnp.float32)


def _to_col(row):
    n = row.shape[1]
    return jax.lax.dot_general(_eye(n), row, (((1,), (1,)), ((), ())),
                               precision=jax.lax.Precision.HIGHEST,
                               preferred_element_type=jnp.float32)


def _score_body(pe_ref, wq_ref, bq_ref, enc_ref, wk_ref,
                sc_ref, m_ref, l_ref, ssum_ref, qp_scr):
    b = pl.program_id(0)
    s = pl.program_id(1)
    scale = 1.0 / math.sqrt(DH)

    @pl.when((b == 0) & (s == 0))
    def _build_qp():
        qp = jax.lax.dot_general(pe_ref[...], wq_ref[...],
                                 (((1,), (1,)), ((), ())),
                                 preferred_element_type=jnp.float32
                                 ) + bq_ref[...]
        lanes = jax.lax.broadcasted_iota(jnp.int32, (1, D_MODEL), 1)
        blocks = []
        for h in range(N_HEADS):
            msk = ((lanes >= h * DH)
                   & (lanes < (h + 1) * DH)).astype(jnp.float32)
            blocks.append(qp * msk)
        qp_scr[...] = jnp.concatenate(blocks, axis=0)

    enc = enc_ref[0]
    k_blk = jax.lax.dot_general(enc, wk_ref[...], (((1,), (1,)), ((), ())),
                                preferred_element_type=jnp.float32)
    scores = jax.lax.dot_general(k_blk, qp_scr[...], (((1,), (1,)), ((), ())),
                                 preferred_element_type=jnp.float32)
    sc_ref[0] = scores
    blk_max = jnp.max(scores, axis=0, keepdims=True)
    blk_sum = jnp.sum(scores, axis=0, keepdims=True)

    @pl.when(s == 0)
    def _init():
        m_ref[0] = blk_max
        l_ref[0] = jnp.sum(jnp.exp((scores - blk_max) * scale),
                           axis=0, keepdims=True)
        ssum_ref[0] = blk_sum

    @pl.when(s != 0)
    def _update():
        m_old = m_ref[0]
        m_new = jnp.maximum(m_old, blk_max)
        alpha = jnp.exp((m_old - m_new) * scale)
        p = jnp.exp((scores - m_new) * scale)
        m_ref[0] = m_new
        l_ref[0] = l_ref[0] * alpha + jnp.sum(p, axis=0, keepdims=True)
        ssum_ref[0] = ssum_ref[0] + blk_sum


def _attn_body(enc_ref, sc_ref, m_ref, l_ref, wv_ref, bv_ref,
               vsum_ref, pacc_ref):
    s = pl.program_id(1)
    enc = enc_ref[0]
    v_blk = jax.lax.dot_general(enc, wv_ref[...], (((1,), (1,)), ((), ())),
                                preferred_element_type=jnp.float32) + bv_ref[...]
    scale = 1.0 / math.sqrt(DH)
    w = jnp.exp((sc_ref[0] - m_ref[0]) * scale) / l_ref[0]
    vsum = jnp.sum(v_blk, axis=0, keepdims=True)
    pacc = jax.lax.dot_general(w, v_blk, (((0,), (0,)), ((), ())),
                               preferred_element_type=jnp.float32)

    @pl.when(s == 0)
    def _init():
        vsum_ref[0] = vsum
        pacc_ref[0] = pacc

    @pl.when(s != 0)
    def _update():
        vsum_ref[0] = vsum_ref[0] + vsum
        pacc_ref[0] = pacc_ref[0] + pacc


NG = B_SZ * N_HEADS
GQ = 32
NCHUNK = NG // 16


def _sc_lazy_select(m_flat, ssum_flat):
    mesh = plsc.VectorSubcoreMesh(core_axis_name="c", subcore_axis_name="s")
    n_in = NCHUNK * GQ * 16
    n_slot = GQ * 16

    @functools.partial(
        pl.kernel, mesh=mesh,
        out_type=jax.ShapeDtypeStruct((32 * n_slot,), jnp.float32),
        scratch_types=[
            pltpu.VMEM((n_in,), jnp.float32),
            pltpu.VMEM((n_in,), jnp.float32),
            pltpu.VMEM((n_slot,), jnp.float32),
            pltpu.VMEM((n_slot,), jnp.float32),
        ],
    )
    def lazy_kernel(m_hbm, s_hbm, out_hbm, m_v, s_v, mst_v, out_v):
        wid = jax.lax.axis_index("s") * 2 + jax.lax.axis_index("c")
        chunk = jax.lax.rem(wid, NCHUNK)
        base = chunk * n_slot
        pltpu.sync_copy(m_hbm, m_v)
        pltpu.sync_copy(s_hbm, s_v)
        inv_s = 1.0 / float(S_LEN)
        for i in range(HORIZON):
            mst_v[pl.ds(i * 16, 16)] = (m_v[pl.ds(base + i * 16, 16)]
                                        - s_v[pl.ds(base + i * 16, 16)] * inv_s)
        zero16 = jnp.zeros((16,), jnp.float32)
        one16 = jnp.ones((16,), jnp.float32)
        for i in range(HORIZON):
            mi = mst_v[pl.ds(i * 16, 16)]
            rank = zero16
            for j in range(HORIZON):
                if j == i:
                    continue
                mj = mst_v[pl.ds(j * 16, 16)]
                beats = (mj >= mi) if j < i else (mj > mi)
                rank = rank + jnp.where(beats, one16, zero16)
            out_v[pl.ds(i * 16, 16)] = jnp.where(rank >= float(U_TOP),
                                                 one16, zero16)
        for i in range(HORIZON, GQ):
            out_v[pl.ds(i * 16, 16)] = zero16
        pltpu.sync_copy(out_v, out_hbm.at[pl.ds(wid * n_slot, n_slot)])

    return lazy_kernel(m_flat, ssum_flat)


def _layer_norm(x, g, b, eps=1e-5):
    m = jnp.mean(x, axis=-1, keepdims=True)
    v = jnp.mean((x - m) ** 2, axis=-1, keepdims=True)
    return (x - m) / jnp.sqrt(v + eps) * g + b


def _tail_body(lazy_ref, m_ref, ssum_ref, vsum_ref, pacc_ref,
               wo_ref, bo_ref, w1_ref, b1_ref, w2_ref, b2_ref,
               g1_ref, be1_ref, g2_ref, be2_ref, wout_ref,
               fc_ref, y_ref):
    del m_ref, ssum_ref
    lazy3 = jnp.concatenate([_to_col(lazy_ref[b])[None] for b in range(B_SZ)],
                            axis=0)

    ctx_all = pacc_ref[...]

    vmean3 = jnp.concatenate([vsum_ref[b][None] for b in range(B_SZ)],
                             axis=0) / S_LEN

    lanes = jax.lax.broadcasted_iota(jnp.int32, (1, 1, D_MODEL), 2)
    attn = jnp.zeros((B_SZ, HQ, D_MODEL), jnp.float32)
    for h in range(N_HEADS):
        mask = ((lanes >= h * DH) & (lanes < (h + 1) * DH)).astype(jnp.float32)
        act = ctx_all[:, h * HQ:(h + 1) * HQ, :]
        lz = lazy3[:, h * HQ:(h + 1) * HQ, :]
        sel = act * (1.0 - lz) + vmean3 * lz
        attn = attn + sel * mask
    attn = attn.reshape(B_SZ * HQ, D_MODEL)

    proj = jax.lax.dot_general(attn, wo_ref[...], (((1,), (1,)), ((), ())),
                               preferred_element_type=jnp.float32) + bo_ref[...]
    x = _layer_norm(proj + proj, g1_ref[...], be1_ref[...])
    h1 = jax.lax.dot_general(x, w1_ref[...], (((1,), (1,)), ((), ())),
                             preferred_element_type=jnp.float32) + b1_ref[...]
    h1 = jnp.maximum(h1, 0.0)
    ff = jax.lax.dot_general(h1, w2_ref[...], (((1,), (1,)), ((), ())),
                             preferred_element_type=jnp.float32) + b2_ref[...]
    y = _layer_norm(x + ff, g2_ref[...], be2_ref[...])
    fc = jax.lax.dot_general(y, wout_ref[...], (((1,), (1,)), ((), ())),
                             preferred_element_type=jnp.float32)
    y_ref[...] = y
    fc_ref[...] = fc


def kernel(encoder_output, Wq, bq, Wk, bk, Wv, bv, Wo, bo,
           W1, b1, W2, b2, g1, be1, g2, be2, Wout, bout):
    pe_pad = jnp.asarray(_make_pe_pad())

    def to_chunks(x):
        g = jnp.pad(x.reshape(NG, HQ), ((0, 0), (0, GQ - HQ)))
        return g.T.reshape(GQ, NCHUNK, 16).transpose(1, 0, 2)

    stats_shape = jax.ShapeDtypeStruct((B_SZ, 1, NQ), jnp.float32)
    sc_hbm, m_s, l_s, ssum_s = pl.pallas_call(
        _score_body,
        grid=(B_SZ, NS),
        in_specs=[
            pl.BlockSpec((HQ, D_MODEL), lambda b, s: (0, 0)),
            pl.BlockSpec((D_MODEL, D_MODEL), lambda b, s: (0, 0)),
            pl.BlockSpec((D_MODEL,), lambda b, s: (0,)),
            pl.BlockSpec((1, S_BLK, D_MODEL), lambda b, s: (b, s, 0)),
            pl.BlockSpec((D_MODEL, D_MODEL), lambda b, s: (0, 0)),
        ],
        out_specs=[
            pl.BlockSpec((1, S_BLK, NQ), lambda b, s: (b, s, 0)),
            pl.BlockSpec((1, 1, NQ), lambda b, s: (b, 0, 0)),
            pl.BlockSpec((1, 1, NQ), lambda b, s: (b, 0, 0)),
            pl.BlockSpec((1, 1, NQ), lambda b, s: (b, 0, 0)),
        ],
        out_shape=[
            jax.ShapeDtypeStruct((B_SZ, S_LEN, NQ), jnp.float32),
            stats_shape, stats_shape, stats_shape,
        ],
        scratch_shapes=[pltpu.VMEM((NQ, D_MODEL), jnp.float32)],
    )(pe_pad, Wq, bq, encoder_output, Wk)

    lazy_flat = _sc_lazy_select(to_chunks(m_s).reshape(-1),
                                to_chunks(ssum_s).reshape(-1))
    lazy_chunks = lazy_flat[:NCHUNK * GQ * 16].reshape(NCHUNK, GQ, 16)
    lazy_row = (lazy_chunks.transpose(1, 0, 2).reshape(GQ, NG)
                .T[:, :HQ].reshape(B_SZ, 1, NQ))

    vsum_s, pacc_s = pl.pallas_call(
        _attn_body,
        grid=(B_SZ, NS),
        in_specs=[
            pl.BlockSpec((1, S_BLK, D_MODEL), lambda b, s: (b, s, 0)),
            pl.BlockSpec((1, S_BLK, NQ), lambda b, s: (b, s, 0)),
            pl.BlockSpec((1, 1, NQ), lambda b, s: (b, 0, 0)),
            pl.BlockSpec((1, 1, NQ), lambda b, s: (b, 0, 0)),
            pl.BlockSpec((D_MODEL, D_MODEL), lambda b, s: (0, 0)),
            pl.BlockSpec((D_MODEL,), lambda b, s: (0,)),
        ],
        out_specs=[
            pl.BlockSpec((1, 1, D_MODEL), lambda b, s: (b, 0, 0)),
            pl.BlockSpec((1, NQ, D_MODEL), lambda b, s: (b, 0, 0)),
        ],
        out_shape=[
            jax.ShapeDtypeStruct((B_SZ, 1, D_MODEL), jnp.float32),
            jax.ShapeDtypeStruct((B_SZ, NQ, D_MODEL), jnp.float32),
        ],
    )(encoder_output, sc_hbm, m_s, l_s, Wv, bv)

    fc_pad, y_pad = pl.pallas_call(
        _tail_body,
        out_shape=[
            jax.ShapeDtypeStruct((B_SZ * HQ, 8), jnp.float32),
            jax.ShapeDtypeStruct((B_SZ * HQ, D_MODEL), jnp.float32),
        ],
    )(lazy_row,
      m_s, ssum_s, vsum_s, pacc_s,
      Wo, bo, W1, b1, W2, b2, g1, be1, g2, be2,
      jnp.pad(Wout, ((0, 7), (0, 0))))

    y = y_pad.reshape(B_SZ, HQ, D_MODEL)[:, :HORIZON, :]
    forecasts = fc_pad[:, 0].reshape(B_SZ, HQ)[:, :HORIZON] + bout
    return forecasts, y
